# Initial kernel scaffold; baseline (speedup 1.0000x reference)
import jax
import jax.numpy as jnp
from jax.experimental import pallas as pl


def kernel(x, edge_index, supermask, Wx1, bx1, Wg, a_src, a_dst, bg, Wz1, bz1):
    raise NotImplementedError("write your pallas kernel here")


# speedup vs baseline: 54.1907x; 54.1907x over previous
import functools
import jax
import jax.numpy as jnp
from jax import lax
from jax.experimental import pallas as pl
from jax.experimental.pallas import tpu as pltpu
from jax.experimental.pallas import tpu_sc as plsc

N = 10000
E = 320000
LW = 16
GW = 192
ROW = 128
NWORK = 32
EPW = E // NWORK
CH = 80
NCH = EPW // CH
RPS = 632
TILE = 1000


def _tc1_body(x_ref, w_ref, b_ref, ba_ref, bb_ref, ta_ref, tb_ref):
    h0 = jax.nn.sigmoid(
        jnp.dot(x_ref[...], w_ref[...], preferred_element_type=jnp.float32)
        + b_ref[...])
    ta_ref[...] = jnp.dot(h0, ba_ref[...], preferred_element_type=jnp.float32)
    tb_ref[...] = jnp.dot(h0, bb_ref[...], preferred_element_type=jnp.float32)


def _tc2_body(pa_ref, pb_ref, ta_ref, tb_ref, c_ref, o_ref):
    pa = pa_ref[0] + pa_ref[1]
    pb = pb_ref[0] + pb_ref[1]
    z = ta_ref[:, 0:LW] + ta_ref[:, LW:2 * LW]
    exs = jnp.exp(jnp.maximum(z, 0.2 * z))
    st = pa[:, 0:LW] + exs
    q = jnp.concatenate([pa[:, 32:ROW], pb[:, 32:ROW]], axis=1)
    g = jnp.concatenate([ta_ref[:, 32:ROW], tb_ref[:, 32:ROW]], axis=1)
    tot = jnp.zeros((TILE, 32), jnp.float32)
    for i in range(6):
        qi = q[:, 32 * i: 32 * (i + 1)] + exs[:, i:i + 1] * \
            g[:, 32 * i: 32 * (i + 1)]
        tot = tot + qi / (st[:, i:i + 1] + 1e-16)
    o_ref[...] = jax.nn.sigmoid(tot * (1.0 / 6.0) + c_ref[...])


def _sc_edges(tbla, tblb, srcidx, dstidx, zeros, out,
              src_v, dst_v, srow_v, arow_v, scr_v, acc_sh):
    cid = lax.axis_index("c")
    sid = lax.axis_index("s")
    wid = cid * 16 + sid
    r0 = sid * RPS
    base = wid * EPW
    last = N - 15 * RPS

    for p in range(2):
        tbl = tbla if p == 0 else tblb

        @pl.when(sid < 15)
        def _():
            pltpu.sync_copy(zeros.at[pl.ds(r0, RPS)],
                            acc_sh.at[pl.ds(r0, RPS)])

        @pl.when(sid == 15)
        def _():
            pltpu.sync_copy(zeros.at[pl.ds(15 * RPS, last)],
                            acc_sh.at[pl.ds(15 * RPS, last)])

        plsc.subcore_barrier()

        def chunk(j, carry):
            off = base + j * CH
            pltpu.sync_copy(srcidx.at[pl.ds(off, CH)], src_v)
            pltpu.sync_copy(dstidx.at[pl.ds(off, CH)], dst_v)
            pltpu.sync_copy(tbl.at[src_v], srow_v)
            pltpu.sync_copy(tbl.at[dst_v], arow_v)

            def edge(k, c2):
                a = srow_v[k, pl.ds(0, LW)]
                b = arow_v[k, pl.ds(LW, LW)]
                e = a + b
                ex = jnp.exp(jnp.maximum(e, 0.2 * e))
                scr_v[k, pl.ds(0, LW)] = ex
                scr_v[k, pl.ds(LW, LW)] = jnp.zeros((16,), jnp.float32)
                for h in range(6):
                    o = 32 + 16 * h
                    exi = jnp.full((16,), ex[3 * p + h // 2],
                                   dtype=jnp.float32)
                    scr_v[k, pl.ds(o, 16)] = exi * srow_v[k, pl.ds(o, 16)]
                return c2

            lax.fori_loop(0, CH, edge, 0)
            pltpu.sync_copy(scr_v, acc_sh.at[dst_v], add=True)
            return carry

        lax.fori_loop(0, NCH, chunk, 0)
        plsc.subcore_barrier()

        @pl.when(sid < 15)
        def _():
            pltpu.sync_copy(acc_sh.at[pl.ds(r0, RPS)],
                            out.at[p, cid, pl.ds(r0, RPS)])

        @pl.when(sid == 15)
        def _():
            pltpu.sync_copy(acc_sh.at[pl.ds(15 * RPS, last)],
                            out.at[p, cid, pl.ds(15 * RPS, last)])

        plsc.subcore_barrier()


def kernel(x, edge_index, supermask, Wx1, bx1, Wg, a_src, a_dst, bg, Wz1, bz1):
    f32 = jnp.float32
    vs = jnp.einsum('ikh,ik->hi', Wg, a_src)
    vd = jnp.einsum('ikh,ik->hi', Wg, a_dst)
    vs = jnp.pad(vs, ((0, 0), (0, LW - 6)))
    vd = jnp.pad(vd, ((0, 0), (0, LW - 6)))
    ut = jnp.einsum('ck,ikh->hic', Wz1, Wg).reshape(64, GW)
    ba = jnp.concatenate([vs, vd, ut[:, 0:96]], axis=1)
    bb = jnp.concatenate([vs, vd, ut[:, 96:192]], axis=1)
    cvec = (jnp.mean(bg, axis=0) @ Wz1.T + bz1).reshape(1, 32)

    grid = N // TILE
    tbla, tblb = pl.pallas_call(
        _tc1_body,
        grid=(grid,),
        in_specs=[
            pl.BlockSpec((TILE, 128), lambda i: (i, 0)),
            pl.BlockSpec((128, 64), lambda i: (0, 0)),
            pl.BlockSpec((1, 64), lambda i: (0, 0)),
            pl.BlockSpec((64, ROW), lambda i: (0, 0)),
            pl.BlockSpec((64, ROW), lambda i: (0, 0)),
        ],
        out_specs=[
            pl.BlockSpec((TILE, ROW), lambda i: (i, 0)),
            pl.BlockSpec((TILE, ROW), lambda i: (i, 0)),
        ],
        out_shape=[
            jax.ShapeDtypeStruct((N, ROW), f32),
            jax.ShapeDtypeStruct((N, ROW), f32),
        ],
    )(x, Wx1.T, bx1.reshape(1, 64), ba, bb)

    sc_call = functools.partial(
        pl.kernel,
        mesh=plsc.VectorSubcoreMesh(core_axis_name="c", subcore_axis_name="s"),
        out_type=jax.ShapeDtypeStruct((2, 2, N, ROW), f32),
        scratch_types=[
            pltpu.VMEM((CH,), jnp.int32),
            pltpu.VMEM((CH,), jnp.int32),
            pltpu.VMEM((CH, ROW), f32),
            pltpu.VMEM((CH, ROW), f32),
            pltpu.VMEM((CH, ROW), f32),
            pltpu.VMEM_SHARED((N, ROW), f32),
        ],
    )(_sc_edges)
    zeros = jnp.zeros((N, ROW), f32)
    scout = sc_call(tbla, tblb, edge_index[0], edge_index[1], zeros)

    out = pl.pallas_call(
        _tc2_body,
        grid=(grid,),
        in_specs=[
            pl.BlockSpec((2, TILE, ROW), lambda i: (0, i, 0)),
            pl.BlockSpec((2, TILE, ROW), lambda i: (0, i, 0)),
            pl.BlockSpec((TILE, ROW), lambda i: (i, 0)),
            pl.BlockSpec((TILE, ROW), lambda i: (i, 0)),
            pl.BlockSpec((1, 32), lambda i: (0, 0)),
        ],
        out_specs=pl.BlockSpec((TILE, 32), lambda i: (i, 0)),
        out_shape=jax.ShapeDtypeStruct((N, 32), f32),
    )(scout[0], scout[1], tbla, tblb, cvec)
    return out
